# Initial kernel scaffold; baseline (speedup 1.0000x reference)
import jax
import jax.numpy as jnp
from jax.experimental import pallas as pl


def kernel(qlen, klen, W):
    raise NotImplementedError("write your pallas kernel here")


# speedup vs baseline: 1.9669x; 1.9669x over previous
import functools
import math

import jax
import jax.numpy as jnp
from jax import lax
from jax.experimental import pallas as pl
from jax.experimental.pallas import tpu as pltpu
from jax.experimental.pallas import tpu_sc as plsc

NUM_BUCKETS = 32
MAX_DISTANCE = 128
NUM_HEADS = 16
QLEN = 2048
KLEN = 2048

VLONG = 4224
TBL = 3968
NUM_TILES = 32
INFLIGHT = 8


def _table_body(delta_ref, w_ref, out_ref):
    c = pl.program_id(0)
    delta = delta_ref[0]
    max_exact = NUM_BUCKETS // 2
    n = lax.broadcasted_iota(jnp.int32, (1, VLONG), 1) + 8 * c
    d = n - (QLEN - 1) + delta
    rp = -jnp.minimum(d, jnp.zeros_like(d))
    is_small = rp < max_exact
    val_large = max_exact + (
        jnp.log(rp.astype(jnp.float32) / max_exact + 1.0)
        / math.log(MAX_DISTANCE / max_exact)
        * (NUM_BUCKETS - max_exact)
    ).astype(jnp.int32)
    val_large = jnp.minimum(val_large, jnp.full_like(val_large, NUM_BUCKETS - 1))
    bucket = jnp.where(is_small, rp, val_large)
    b_iota = lax.broadcasted_iota(jnp.int32, (NUM_BUCKETS, VLONG), 0)
    onehot = (bucket == b_iota).astype(jnp.float32)
    vc = lax.dot_general(w_ref[...], onehot, (((0,), (0,)), ((), ())),
                         preferred_element_type=jnp.float32)
    for a in range(8):
        out_ref[:, 0, a, :] = lax.slice(vc, (0, 7 - a), (NUM_HEADS, 7 - a + TBL))


def _build_tables(delta, W):
    return pl.pallas_call(
        _table_body,
        grid=(16,),
        out_shape=jax.ShapeDtypeStruct((NUM_HEADS, 16, 8, TBL), jnp.float32),
        in_specs=[
            pl.BlockSpec(memory_space=pltpu.SMEM),
            pl.BlockSpec((NUM_BUCKETS, NUM_HEADS), lambda c: (0, 0)),
        ],
        out_specs=pl.BlockSpec((NUM_HEADS, 1, 8, TBL), lambda c: (0, c, 0, 0)),
    )(delta, W)


@functools.cache
def _make_expand():
    @functools.partial(
        pl.kernel,
        mesh=plsc.VectorSubcoreMesh(core_axis_name="c", subcore_axis_name="s"),
        out_type=jax.ShapeDtypeStruct((NUM_HEADS, QLEN, KLEN), jnp.float32),
        scratch_types=[pltpu.SemaphoreType.DMA],
    )
    def _expand(t_hbm, out_hbm, sem):
        wid = lax.axis_index("s") * 2 + lax.axis_index("c")
        h = wid // 2
        half = wid % 2
        base = half * (QLEN // 2)
        nblocks = (QLEN // 2) // 8

        def group(g, carry):
            copies = []
            for b in range(INFLIGHT):
                i0 = base + (g * INFLIGHT + b) * 8
                q = (QLEN - 8) - i0
                c = lax.shift_right_logical(lax.bitwise_and(q, 127), 3)
                colq = lax.bitwise_and(q, -128)
                col = pl.multiple_of(colq, 128)
                copies.append(pltpu.async_copy(
                    t_hbm.at[h, c, :, pl.ds(col, KLEN)],
                    out_hbm.at[h, pl.ds(pl.multiple_of(i0, 8), 8), :],
                    sem))
            for cp in copies:
                cp.wait()
            return carry

        lax.fori_loop(0, nblocks // INFLIGHT, group, 0)

    return _expand


def kernel(qlen, klen, W):
    delta = (jnp.asarray(klen, jnp.int32) - jnp.asarray(qlen, jnp.int32)).reshape((1,))
    tbl = _build_tables(delta, W.astype(jnp.float32))
    return _make_expand()(tbl)
